# baseline (speedup 1.0000x reference)
import functools

import jax
import jax.numpy as jnp
from jax import lax
from jax.experimental import pallas as pl
from jax.experimental.pallas import tpu as pltpu
from jax.experimental.pallas import tpu_sc as plsc

M = 16
TOPK = 4
NC, NS, L = 2, 16, 16
NW = NC * NS
D2_THRESH = 4.000000238418579


def _sc_body(xy_hbm, out_hbm, buf, obuf):
    N = 4
    HW = 65536
    C = buf.shape[1]
    ppw = HW // NW
    nchunk = ppw // C
    wid = lax.axis_index("s") * NC + lax.axis_index("c")

    def chunk_body(t, carry):
        n = t // nchunk
        ch = t % nchunk
        base = wid * ppw + ch * C
        pltpu.sync_copy(xy_hbm.at[n, :, pl.ds(base, C)], buf)

        def group(g, carry2):
            col = g * L
            x = [buf[m, pl.ds(col, L)] for m in range(M)]
            y = [buf[M + m, pl.ds(col, L)] for m in range(M)]
            zero = jnp.zeros((L,), jnp.int32)
            sup = [None] * M
            keep = [None] * M
            for m in range(M):
                km = (jnp.ones((L,), jnp.int32) if sup[m] is None
                      else 1 - sup[m])
                keep[m] = km
                for j in range(m + 1, M):
                    dx = x[j] - x[m]
                    dy = y[j] - y[m]
                    d2 = dx * dx + dy * dy
                    hit = jnp.where(d2 <= D2_THRESH, km, 0)
                    sup[j] = hit if sup[j] is None else sup[j] | hit
            cnt = zero
            kept_before = [None] * M
            for m in range(M):
                kept_before[m] = cnt
                cnt = cnt + keep[m]
            outs = [zero, zero, zero, zero]
            for m in range(M):
                rank_m = jnp.where(keep[m] == 1, kept_before[m],
                                   cnt + (m - kept_before[m]))
                for k in range(TOPK):
                    outs[k] = outs[k] | jnp.where(rank_m == k, jnp.int32(m), 0)
            for k in range(TOPK):
                obuf[k, pl.ds(col, L)] = outs[k]
            return carry2

        lax.fori_loop(0, C // L, group, 0)
        pltpu.sync_copy(obuf, out_hbm.at[n, :, pl.ds(base, C)])
        return carry

    lax.fori_loop(0, N * nchunk, chunk_body, 0)


def _make_sc_call(N, HW, C=512, interpret=False):
    mesh = plsc.VectorSubcoreMesh(core_axis_name="c", subcore_axis_name="s",
                                  num_cores=NC, num_subcores=NS)
    return functools.partial(
        pl.kernel,
        out_type=jax.ShapeDtypeStruct((N, TOPK, HW), jnp.int32),
        mesh=mesh,
        interpret=interpret,
        scratch_types=[
            pltpu.VMEM((2 * M, C), jnp.float32),
            pltpu.VMEM((TOPK, C), jnp.int32),
        ],
    )(_sc_body)


@jax.jit
def kernel(coords_grid, anchor_P):
    N, M_, _, H, W = coords_grid.shape
    HW = H * W
    cg = jnp.transpose(coords_grid, (0, 2, 3, 4, 1)).reshape(N, 3, HW, M_)
    proj = jnp.einsum('nij,njkm->nikm', anchor_P, cg)
    x_2d = proj[:, :2] / proj[:, 2:3]
    xy = jnp.transpose(x_2d, (0, 1, 3, 2)).reshape(N, 2 * M_, HW)
    out = _make_sc_call(N, HW)(xy)
    out = out.reshape(N, TOPK, H, W)
    return jnp.transpose(out, (0, 2, 3, 1)).astype(jnp.int64)


# speedup vs baseline: 1.3511x; 1.3511x over previous
import functools

import jax
import jax.numpy as jnp
from jax import lax
from jax.experimental import pallas as pl
from jax.experimental.pallas import tpu as pltpu
from jax.experimental.pallas import tpu_sc as plsc

M = 16
TOPK = 4
HB = 32
H_SC = 64
NC, NS, L = 2, 16, 16
NW = NC * NS
D2_THRESH = 4.000000238418579


def _nms_body_tc(pj_ref, out_ref):
    x = []
    y = []
    for m in range(M):
        px = pj_ref[0, m, 0]
        py = pj_ref[0, m, 1]
        pz = pj_ref[0, m, 2]
        x.append(px / pz)
        y.append(py / pz)

    shape = x[0].shape
    ones = jnp.ones(shape, dtype=jnp.bool_)

    supp = [None] * M
    keep = [None] * M
    for m in range(M):
        km = ones if supp[m] is None else jnp.logical_not(supp[m])
        keep[m] = km
        for j in range(m + 1, M):
            dx = x[j] - x[m]
            dy = y[j] - y[m]
            d = jnp.sqrt(dx * dx + dy * dy)
            c = jnp.logical_and(km, d <= 2.0)
            supp[j] = c if supp[j] is None else jnp.logical_or(supp[j], c)

    zero = jnp.zeros(shape, dtype=jnp.int32)
    cnt = zero
    kept_before = [None] * M
    for m in range(M):
        kept_before[m] = cnt
        cnt = cnt + keep[m].astype(jnp.int32)
    for k in range(TOPK):
        acc = zero
        for m in range(M):
            rank_m = jnp.where(keep[m], kept_before[m],
                               cnt + (m - kept_before[m]))
            acc = acc | jnp.where(rank_m == k, jnp.int32(m), 0)
        out_ref[0, k] = acc


def _sc_body(xy_hbm, out_hbm, buf, obuf):
    N = out_hbm.shape[0]
    HW = out_hbm.shape[2]
    C = buf.shape[1]
    ppw = HW // NW
    nchunk = ppw // C
    wid = lax.axis_index("s") * NC + lax.axis_index("c")

    def chunk_body(t, carry):
        n = t // nchunk
        ch = t % nchunk
        base = wid * ppw + ch * C
        pltpu.sync_copy(xy_hbm.at[n, :, pl.ds(base, C)], buf)

        def group(g, carry2):
            col = g * L
            x = [buf[m, pl.ds(col, L)] for m in range(M)]
            y = [buf[M + m, pl.ds(col, L)] for m in range(M)]
            zero = jnp.zeros((L,), jnp.int32)
            sup = [None] * M
            keep = [None] * M
            for m in range(M):
                km = (jnp.ones((L,), jnp.int32) if sup[m] is None
                      else 1 - sup[m])
                keep[m] = km
                for j in range(m + 1, M):
                    dx = x[j] - x[m]
                    dy = y[j] - y[m]
                    d2 = dx * dx + dy * dy
                    hit = jnp.where(d2 <= D2_THRESH, km, 0)
                    sup[j] = hit if sup[j] is None else sup[j] | hit
            cnt = zero
            kept_before = [None] * M
            for m in range(M):
                kept_before[m] = cnt
                cnt = cnt + keep[m]
            outs = [zero, zero, zero, zero]
            for m in range(M):
                rank_m = jnp.where(keep[m] == 1, kept_before[m],
                                   cnt + (m - kept_before[m]))
                for k in range(TOPK):
                    outs[k] = outs[k] | jnp.where(rank_m == k, jnp.int32(m), 0)
            for k in range(TOPK):
                obuf[k, pl.ds(col, L)] = outs[k]
            return carry2

        lax.fori_loop(0, C // L, group, 0)
        pltpu.sync_copy(obuf, out_hbm.at[n, :, pl.ds(base, C)])
        return carry

    lax.fori_loop(0, N * nchunk, chunk_body, 0)


def _make_sc_call(N, HW, C=512):
    mesh = plsc.VectorSubcoreMesh(core_axis_name="c", subcore_axis_name="s",
                                  num_cores=NC, num_subcores=NS)
    return functools.partial(
        pl.kernel,
        out_type=jax.ShapeDtypeStruct((N, TOPK, HW), jnp.int32),
        mesh=mesh,
        scratch_types=[
            pltpu.VMEM((2 * M, C), jnp.float32),
            pltpu.VMEM((TOPK, C), jnp.int32),
        ],
    )(_sc_body)


@jax.jit
def kernel(coords_grid, anchor_P):
    N, M_, _, H, W = coords_grid.shape
    pj = jnp.einsum('nij,nmjhw->nmihw', anchor_P, coords_grid)

    h_tc = H - H_SC
    out_tc = pl.pallas_call(
        _nms_body_tc,
        grid=(N, h_tc // HB),
        in_specs=[
            pl.BlockSpec((1, M_, 3, HB, W), lambda n, h: (n, 0, 0, h, 0)),
        ],
        out_specs=pl.BlockSpec((1, TOPK, HB, W), lambda n, h: (n, 0, h, 0)),
        out_shape=jax.ShapeDtypeStruct((N, TOPK, h_tc, W), jnp.int32),
    )(pj[:, :, :, :h_tc])

    hw_sc = H_SC * W
    pjs = pj[:, :, :, h_tc:].reshape(N, M_, 3, hw_sc)
    xy2 = pjs[:, :, :2] / pjs[:, :, 2:3]
    xy = jnp.transpose(xy2, (0, 2, 1, 3)).reshape(N, 2 * M_, hw_sc)
    out_sc = _make_sc_call(N, hw_sc)(xy).reshape(N, TOPK, H_SC, W)

    out = jnp.concatenate([out_tc, out_sc], axis=2)
    return jnp.transpose(out, (0, 2, 3, 1)).astype(jnp.int64)


# speedup vs baseline: 1.3527x; 1.0012x over previous
import functools

import jax
import jax.numpy as jnp
from jax import lax
from jax.experimental import pallas as pl
from jax.experimental.pallas import tpu as pltpu
from jax.experimental.pallas import tpu_sc as plsc

M = 16
TOPK = 4
HB = 32
H_SC = 64
NC, NS, L = 2, 16, 16
NW = NC * NS
D2_THRESH = 4.000000238418579


def _nms_body_tc(pj_ref, out_ref):
    x = []
    y = []
    for m in range(M):
        px = pj_ref[0, m, 0]
        py = pj_ref[0, m, 1]
        pz = pj_ref[0, m, 2]
        x.append(px / pz)
        y.append(py / pz)

    shape = x[0].shape
    ones = jnp.ones(shape, dtype=jnp.bool_)

    supp = [None] * M
    keep = [None] * M
    for m in range(M):
        km = ones if supp[m] is None else jnp.logical_not(supp[m])
        keep[m] = km
        for j in range(m + 1, M):
            dx = x[j] - x[m]
            dy = y[j] - y[m]
            d = jnp.sqrt(dx * dx + dy * dy)
            c = jnp.logical_and(km, d <= 2.0)
            supp[j] = c if supp[j] is None else jnp.logical_or(supp[j], c)

    zero = jnp.zeros(shape, dtype=jnp.int32)
    cnt = zero
    kept_before = [None] * M
    for m in range(M):
        kept_before[m] = cnt
        cnt = cnt + keep[m].astype(jnp.int32)
    for k in range(TOPK):
        acc = zero
        for m in range(M):
            rank_m = jnp.where(keep[m], kept_before[m],
                               cnt + (m - kept_before[m]))
            acc = acc | jnp.where(rank_m == k, jnp.int32(m), 0)
        out_ref[0, k] = acc


def _sc_body(xy_hbm, out_hbm, buf, obuf):
    N = out_hbm.shape[0]
    HW = out_hbm.shape[2]
    C = buf.shape[1]
    ppw = HW // NW
    nchunk = ppw // C
    wid = lax.axis_index("s") * NC + lax.axis_index("c")

    def chunk_body(t, carry):
        n = t // nchunk
        ch = t % nchunk
        base = wid * ppw + ch * C
        pltpu.sync_copy(xy_hbm.at[n, :, pl.ds(base, C)], buf)

        def group(g, carry2):
            col = g * L
            x = [buf[m, pl.ds(col, L)] for m in range(M)]
            y = [buf[M + m, pl.ds(col, L)] for m in range(M)]
            zero = jnp.zeros((L,), jnp.int32)
            sup = [None] * M
            keep = [None] * M
            for m in range(M):
                km = (jnp.ones((L,), jnp.int32) if sup[m] is None
                      else 1 - sup[m])
                keep[m] = km
                for j in range(m + 1, M):
                    dx = x[j] - x[m]
                    dy = y[j] - y[m]
                    d2 = dx * dx + dy * dy
                    hit = jnp.where(d2 <= D2_THRESH, km, 0)
                    sup[j] = hit if sup[j] is None else sup[j] | hit
            cnt = zero
            kept_before = [None] * M
            for m in range(M):
                kept_before[m] = cnt
                cnt = cnt + keep[m]
            outs = [zero, zero, zero, zero]
            for m in range(M):
                rank_m = jnp.where(keep[m] == 1, kept_before[m],
                                   cnt + (m - kept_before[m]))
                for k in range(TOPK):
                    outs[k] = outs[k] | jnp.where(rank_m == k, jnp.int32(m), 0)
            for k in range(TOPK):
                obuf[k, pl.ds(col, L)] = outs[k]
            return carry2

        lax.fori_loop(0, C // L, group, 0)
        pltpu.sync_copy(obuf, out_hbm.at[n, :, pl.ds(base, C)])
        return carry

    lax.fori_loop(0, N * nchunk, chunk_body, 0)


def _make_sc_call(N, HW, C=512):
    mesh = plsc.VectorSubcoreMesh(core_axis_name="c", subcore_axis_name="s",
                                  num_cores=NC, num_subcores=NS)
    return functools.partial(
        pl.kernel,
        out_type=jax.ShapeDtypeStruct((N, TOPK, HW), jnp.int32),
        mesh=mesh,
        scratch_types=[
            pltpu.VMEM((2 * M, C), jnp.float32),
            pltpu.VMEM((TOPK, C), jnp.int32),
        ],
    )(_sc_body)


@jax.jit
def kernel(coords_grid, anchor_P):
    N, M_, _, H, W = coords_grid.shape
    pj = jnp.einsum('nij,nmjhw->nmihw', anchor_P, coords_grid)

    h_tc = H - H_SC
    hw_sc = H_SC * W
    pjs = pj[:, :, :, h_tc:].reshape(N, M_, 3, hw_sc)
    xy2 = pjs[:, :, :2] / pjs[:, :, 2:3]
    xy = jnp.transpose(xy2, (0, 2, 1, 3)).reshape(N, 2 * M_, hw_sc)
    out_sc = _make_sc_call(N, hw_sc)(xy).reshape(N, TOPK, H_SC, W)

    out_tc = pl.pallas_call(
        _nms_body_tc,
        grid=(N, h_tc // HB),
        in_specs=[
            pl.BlockSpec((1, M_, 3, HB, W), lambda n, h: (n, 0, 0, h, 0)),
        ],
        out_specs=pl.BlockSpec((1, TOPK, HB, W), lambda n, h: (n, 0, h, 0)),
        out_shape=jax.ShapeDtypeStruct((N, TOPK, h_tc, W), jnp.int32),
    )(pj[:, :, :, :h_tc])

    out = jnp.concatenate([out_tc, out_sc], axis=2)
    return jnp.transpose(out, (0, 2, 3, 1)).astype(jnp.int64)
